# baseline (speedup 1.0000x reference)
import jax
import jax.numpy as jnp
from jax.experimental import pallas as pl
from jax.experimental.pallas import tpu as pltpu

V = 4096
D = 128
B = 8
S = 32
P = 8
CONC = 5.0
T = 8
SG = S // T
NG = B * SG
NB = 4


def _kernel(idx_ref, seg_ref, pmdv_ref, cw_ref,
            embed_ref, kb_ref, aff_ref, lam_ref, edge_ref,
            out_ref, buf, a_scr, c_scr, g_scr, sem):
    i = pl.program_id(0)

    def issue(gi, slot):
        gb = gi // SG
        @pl.when(seg_ref[gb] > jax.lax.rem(gi, SG) * T)
        def _issue_group():
            for t in range(T):
                for p in range(P):
                    c = idx_ref[(gi * T + t) * P + p]
                    pltpu.make_async_copy(
                        edge_ref.at[pl.ds(c, 1), :],
                        buf.at[slot, p, pl.ds(t, 1), :],
                        sem.at[slot]).start()

    @pl.when(i == 0)
    def _prologue():
        for k in range(NB - 1):
            issue(k, k)
        ctx = jax.lax.dot_general(
            cw_ref[:, :], embed_ref[:, :], (((1,), (0,)), ((), ())),
            preferred_element_type=jnp.float32)
        cn2 = jnp.sum(ctx * ctx, axis=1, keepdims=True)
        cnm = ctx / jnp.maximum(jnp.sqrt(cn2), 1e-8)
        kb = kb_ref[:, :]
        dots = jax.lax.dot_general(
            cnm, kb, (((1,), (1,)), ((), ())),
            preferred_element_type=jnp.float32)
        kn2 = jax.lax.dot_general(
            jnp.ones((1, D), jnp.float32), kb * kb,
            (((1,), (1,)), ((), ())),
            preferred_element_type=jnp.float32)
        lamv = lam_ref[:, :]
        scl = (CONC * lamv) / jnp.maximum(jnp.sqrt(kn2), 1e-8)
        a_scr[:, :] = jnp.abs(dots) * scl
        c_scr[:, :] = CONC * (1.0 - lamv) * aff_ref[:, :]

    gi2 = i + NB - 1

    @pl.when(gi2 < NG)
    def _issue_ahead():
        issue(gi2, jax.lax.rem(gi2, NB))

    slot = jax.lax.rem(i, NB)
    b = i // SG
    segb = seg_ref[b]

    @pl.when(segb > jax.lax.rem(i, SG) * T)
    def _compute():
        for t in range(T):
            for p in range(P):
                pltpu.make_async_copy(
                    edge_ref.at[pl.ds(0, 1), :],
                    buf.at[slot, p, pl.ds(t, 1), :],
                    sem.at[slot]).wait()

        av = a_scr[pl.ds(b, 1), :]
        cv = c_scr[:, :]
        pmd = pmdv_ref[0]
        acc = jnp.zeros((T, V), jnp.float32)
        for p in range(P):
            rows = buf[slot, p]
            e = jnp.exp(rows * av + jnp.where(rows > 0, cv, 0.0))
            ssum = jnp.sum(e, axis=1, keepdims=True)
            colp = pmd[:, p:p + 1]
            acc = acc + e * (colp / ssum)
        g_scr[pl.ds(i * T, T), :] = acc

    @pl.when(segb <= jax.lax.rem(i, SG) * T)
    def _dead_group():
        g_scr[pl.ds(i * T, T), :] = jnp.zeros((T, V), jnp.float32)

    @pl.when(i == NG - 1)
    def _epilogue():
        out_ref[:, :] = jnp.dot(g_scr[:, :], kb_ref[:, :],
                                preferred_element_type=jnp.float32)


@jax.jit
def kernel(concepts, concepts_length, seg_len, embed, embed_kb,
           edge_matrix, affectiveness, lam):
    clen2 = concepts_length.astype(jnp.int32)
    seg = seg_len.astype(jnp.int32)

    valid = jnp.logical_and(
        jnp.arange(P)[None, None, :] < clen2[:, :, None],
        jnp.arange(S)[None, :, None] < seg[:, None, None])
    pmask = valid.astype(jnp.float32)
    pmd = pmask / jnp.maximum(clen2, 1)[:, :, None]

    idx = concepts.astype(jnp.int32).reshape(-1)
    pmdv = pmd.reshape(NG, T, P)

    cw = jnp.zeros((B, V), jnp.float32).at[
        jnp.arange(B)[:, None], concepts.astype(jnp.int32).reshape(B, S * P)
    ].add(pmd.reshape(B, S * P) / seg[:, None].astype(jnp.float32))

    aff2 = affectiveness.reshape(1, V)
    lam2 = lam.reshape(1, V)

    full = lambda i, *_: (0, 0)

    out = pl.pallas_call(
        _kernel,
        grid_spec=pltpu.PrefetchScalarGridSpec(
            num_scalar_prefetch=2,
            grid=(NG,),
            in_specs=[
                pl.BlockSpec((1, T, P), lambda i, *_: (i, 0, 0)),
                pl.BlockSpec((B, V), full),
                pl.BlockSpec((V, D), full),
                pl.BlockSpec((V, D), full),
                pl.BlockSpec((1, V), full),
                pl.BlockSpec((1, V), full),
                pl.BlockSpec(memory_space=pl.ANY),
            ],
            out_specs=pl.BlockSpec((B * S, D), lambda i, *_: (0, 0)),
            scratch_shapes=[
                pltpu.VMEM((NB, P, T, V), jnp.float32),
                pltpu.VMEM((B, V), jnp.float32),
                pltpu.VMEM((1, V), jnp.float32),
                pltpu.VMEM((B * S, V), jnp.float32),
                pltpu.SemaphoreType.DMA((NB,)),
            ],
        ),
        out_shape=jax.ShapeDtypeStruct((B * S, D), jnp.float32),
    )(idx, seg, pmdv, cw, embed, embed_kb, aff2, lam2, edge_matrix)
    return out.reshape(B, S, D)


# speedup vs baseline: 2.1174x; 2.1174x over previous
import jax
import jax.numpy as jnp
from jax.experimental import pallas as pl
from jax.experimental.pallas import tpu as pltpu

V = 4096
D = 128
B = 8
S = 32
P = 8
CONC = 5.0
T = 8
SG = S // T
NG = B * SG
NB = 4


def _kernel(idx_ref, seg_ref, pmds_ref, pmdv_ref,
            embed_ref, kb_ref, aff_ref, lam_ref, edge_ref,
            out_ref, buf, a_scr, c_scr, g_scr, sem):
    i = pl.program_id(0)

    def issue(gi, slot):
        gb = gi // SG
        @pl.when(seg_ref[gb] > jax.lax.rem(gi, SG) * T)
        def _issue_group():
            for t in range(T):
                for p in range(P):
                    c = idx_ref[(gi * T + t) * P + p]
                    pltpu.make_async_copy(
                        edge_ref.at[pl.ds(c, 1), :],
                        buf.at[slot, p, pl.ds(t, 1), :],
                        sem.at[slot]).start()

    @pl.when(i == 0)
    def _prologue():
        for k in range(NB - 1):
            issue(k, k)
        cn_rows = []
        for bb in range(B):
            segb = seg_ref[bb]

            def body(j, acc):
                j2 = 2 * j
                c0 = idx_ref[bb * S * P + j2]
                c1 = idx_ref[bb * S * P + j2 + 1]
                w0 = pmds_ref[bb * S * P + j2]
                w1 = pmds_ref[bb * S * P + j2 + 1]
                return (acc + w0 * embed_ref[pl.ds(c0, 1), :]
                        + w1 * embed_ref[pl.ds(c1, 1), :])

            ctx = jax.lax.fori_loop(0, segb * P // 2, body,
                                    jnp.zeros((1, D), jnp.float32))
            ctx = ctx / segb.astype(jnp.float32)
            nrm = jnp.sqrt(jnp.sum(ctx * ctx))
            cn_rows.append(ctx / jnp.maximum(nrm, 1e-8))
        cnm = jnp.concatenate(cn_rows, axis=0)
        kb = kb_ref[:, :]
        dots = jax.lax.dot_general(
            cnm, kb, (((1,), (1,)), ((), ())),
            preferred_element_type=jnp.float32)
        kn2 = jax.lax.dot_general(
            jnp.ones((1, D), jnp.float32), kb * kb,
            (((1,), (1,)), ((), ())),
            preferred_element_type=jnp.float32)
        lamv = lam_ref[:, :]
        scl = (CONC * lamv) / jnp.maximum(jnp.sqrt(kn2), 1e-8)
        a_scr[:, :] = jnp.abs(dots) * scl
        c_scr[:, :] = CONC * (1.0 - lamv) * aff_ref[:, :]

    gi2 = i + NB - 1

    @pl.when(gi2 < NG)
    def _issue_ahead():
        issue(gi2, jax.lax.rem(gi2, NB))

    slot = jax.lax.rem(i, NB)
    b = i // SG
    segb = seg_ref[b]

    @pl.when(segb > jax.lax.rem(i, SG) * T)
    def _compute():
        for t in range(T):
            for p in range(P):
                pltpu.make_async_copy(
                    edge_ref.at[pl.ds(0, 1), :],
                    buf.at[slot, p, pl.ds(t, 1), :],
                    sem.at[slot]).wait()

        av = a_scr[pl.ds(b, 1), :]
        cv = c_scr[:, :]
        pmd = pmdv_ref[0]
        acc = jnp.zeros((T, V), jnp.float32)
        for p in range(P):
            rows = buf[slot, p]
            e = jnp.exp(rows * av + jnp.where(rows > 0, cv, 0.0))
            ssum = jnp.sum(e, axis=1, keepdims=True)
            colp = pmd[:, p:p + 1]
            acc = acc + e * (colp / ssum)
        g_scr[pl.ds(i * T, T), :] = acc

    @pl.when(segb <= jax.lax.rem(i, SG) * T)
    def _dead_group():
        g_scr[pl.ds(i * T, T), :] = jnp.zeros((T, V), jnp.float32)

    @pl.when(i == NG - 1)
    def _epilogue():
        out_ref[:, :] = jnp.dot(g_scr[:, :], kb_ref[:, :],
                                preferred_element_type=jnp.float32)


@jax.jit
def kernel(concepts, concepts_length, seg_len, embed, embed_kb,
           edge_matrix, affectiveness, lam):
    clen2 = concepts_length.astype(jnp.int32)
    seg = seg_len.astype(jnp.int32)

    valid = jnp.logical_and(
        jnp.arange(P)[None, None, :] < clen2[:, :, None],
        jnp.arange(S)[None, :, None] < seg[:, None, None])
    pmask = valid.astype(jnp.float32)
    pmd = (pmask / jnp.maximum(clen2, 1)[:, :, None]).reshape(-1)

    idx = concepts.astype(jnp.int32).reshape(-1)
    pmdv = pmd.reshape(NG, T, P)

    aff2 = affectiveness.reshape(1, V)
    lam2 = lam.reshape(1, V)

    full = lambda i, *_: (0, 0)

    out = pl.pallas_call(
        _kernel,
        grid_spec=pltpu.PrefetchScalarGridSpec(
            num_scalar_prefetch=3,
            grid=(NG,),
            in_specs=[
                pl.BlockSpec((1, T, P), lambda i, *_: (i, 0, 0)),
                pl.BlockSpec((V, D), full),
                pl.BlockSpec((V, D), full),
                pl.BlockSpec((1, V), full),
                pl.BlockSpec((1, V), full),
                pl.BlockSpec(memory_space=pl.ANY),
            ],
            out_specs=pl.BlockSpec((B * S, D), lambda i, *_: (0, 0)),
            scratch_shapes=[
                pltpu.VMEM((NB, P, T, V), jnp.float32),
                pltpu.VMEM((B, V), jnp.float32),
                pltpu.VMEM((1, V), jnp.float32),
                pltpu.VMEM((B * S, V), jnp.float32),
                pltpu.SemaphoreType.DMA((NB,)),
            ],
        ),
        out_shape=jax.ShapeDtypeStruct((B * S, D), jnp.float32),
    )(idx, seg, pmd, pmdv, embed, embed_kb, aff2, lam2, edge_matrix)
    return out.reshape(B, S, D)


# speedup vs baseline: 2.2652x; 1.0698x over previous
import jax
import jax.numpy as jnp
from jax.experimental import pallas as pl
from jax.experimental.pallas import tpu as pltpu

V = 4096
D = 128
B = 8
S = 32
P = 8
CONC = 5.0
T = 16
SG = S // T
NG = B * SG
NB = 4


def _kernel(idx_ref, seg_ref, pmds_ref, pmdv_ref,
            embed_ref, kb_ref, aff_ref, lam_ref, edge_ref,
            out_ref, buf, a_scr, c_scr, g_scr, sem):
    i = pl.program_id(0)

    def issue(gi, slot):
        gb = gi // SG
        @pl.when(seg_ref[gb] > jax.lax.rem(gi, SG) * T)
        def _issue_group():
            for t in range(T):
                for p in range(P):
                    c = idx_ref[(gi * T + t) * P + p]
                    pltpu.make_async_copy(
                        edge_ref.at[pl.ds(c, 1), :],
                        buf.at[slot, p, pl.ds(t, 1), :],
                        sem.at[slot]).start()

    @pl.when(i == 0)
    def _prologue():
        for k in range(NB - 1):
            issue(k, k)
        cn_rows = []
        for bb in range(B):
            segb = seg_ref[bb]

            def body(j, acc):
                j2 = 2 * j
                c0 = idx_ref[bb * S * P + j2]
                c1 = idx_ref[bb * S * P + j2 + 1]
                w0 = pmds_ref[bb * S * P + j2]
                w1 = pmds_ref[bb * S * P + j2 + 1]
                return (acc + w0 * embed_ref[pl.ds(c0, 1), :]
                        + w1 * embed_ref[pl.ds(c1, 1), :])

            ctx = jax.lax.fori_loop(0, segb * P // 2, body,
                                    jnp.zeros((1, D), jnp.float32))
            ctx = ctx / segb.astype(jnp.float32)
            nrm = jnp.sqrt(jnp.sum(ctx * ctx))
            cn_rows.append(ctx / jnp.maximum(nrm, 1e-8))
        cnm = jnp.concatenate(cn_rows, axis=0)
        kb = kb_ref[:, :]
        dots = jax.lax.dot_general(
            cnm, kb, (((1,), (1,)), ((), ())),
            preferred_element_type=jnp.float32)
        kn2 = jax.lax.dot_general(
            jnp.ones((1, D), jnp.float32), kb * kb,
            (((1,), (1,)), ((), ())),
            preferred_element_type=jnp.float32)
        lamv = lam_ref[:, :]
        scl = (CONC * lamv) / jnp.maximum(jnp.sqrt(kn2), 1e-8)
        a_scr[:, :] = jnp.abs(dots) * scl
        c_scr[:, :] = CONC * (1.0 - lamv) * aff_ref[:, :]

    gi2 = i + NB - 1

    @pl.when(gi2 < NG)
    def _issue_ahead():
        issue(gi2, jax.lax.rem(gi2, NB))

    slot = jax.lax.rem(i, NB)
    b = i // SG
    segb = seg_ref[b]

    @pl.when(segb > jax.lax.rem(i, SG) * T)
    def _compute():
        for t in range(T):
            for p in range(P):
                pltpu.make_async_copy(
                    edge_ref.at[pl.ds(0, 1), :],
                    buf.at[slot, p, pl.ds(t, 1), :],
                    sem.at[slot]).wait()

        av = a_scr[pl.ds(b, 1), :]
        cv = c_scr[:, :]
        pmd = pmdv_ref[0]
        acc = jnp.zeros((T, V), jnp.float32)
        for p in range(P):
            rows = buf[slot, p]
            e = jnp.exp(rows * av + jnp.where(rows > 0, cv, 0.0))
            ssum = jnp.sum(e, axis=1, keepdims=True)
            colp = pmd[:, p:p + 1]
            acc = acc + e * (colp / ssum)
        g_scr[pl.ds(i * T, T), :] = acc

    @pl.when(segb <= jax.lax.rem(i, SG) * T)
    def _dead_group():
        g_scr[pl.ds(i * T, T), :] = jnp.zeros((T, V), jnp.float32)

    @pl.when(i == NG - 1)
    def _epilogue():
        out_ref[:, :] = jnp.dot(g_scr[:, :], kb_ref[:, :],
                                preferred_element_type=jnp.float32)


@jax.jit
def kernel(concepts, concepts_length, seg_len, embed, embed_kb,
           edge_matrix, affectiveness, lam):
    clen2 = concepts_length.astype(jnp.int32)
    seg = seg_len.astype(jnp.int32)

    valid = jnp.logical_and(
        jnp.arange(P)[None, None, :] < clen2[:, :, None],
        jnp.arange(S)[None, :, None] < seg[:, None, None])
    pmask = valid.astype(jnp.float32)
    pmd = (pmask / jnp.maximum(clen2, 1)[:, :, None]).reshape(-1)

    idx = concepts.astype(jnp.int32).reshape(-1)
    pmdv = pmd.reshape(NG, T, P)

    aff2 = affectiveness.reshape(1, V)
    lam2 = lam.reshape(1, V)

    full = lambda i, *_: (0, 0)

    out = pl.pallas_call(
        _kernel,
        grid_spec=pltpu.PrefetchScalarGridSpec(
            num_scalar_prefetch=3,
            grid=(NG,),
            in_specs=[
                pl.BlockSpec((1, T, P), lambda i, *_: (i, 0, 0)),
                pl.BlockSpec((V, D), full),
                pl.BlockSpec((V, D), full),
                pl.BlockSpec((1, V), full),
                pl.BlockSpec((1, V), full),
                pl.BlockSpec(memory_space=pl.ANY),
            ],
            out_specs=pl.BlockSpec((B * S, D), lambda i, *_: (0, 0)),
            scratch_shapes=[
                pltpu.VMEM((NB, P, T, V), jnp.float32),
                pltpu.VMEM((B, V), jnp.float32),
                pltpu.VMEM((1, V), jnp.float32),
                pltpu.VMEM((B * S, V), jnp.float32),
                pltpu.SemaphoreType.DMA((NB,)),
            ],
        ),
        out_shape=jax.ShapeDtypeStruct((B * S, D), jnp.float32),
    )(idx, seg, pmd, pmdv, embed, embed_kb, aff2, lam2, edge_matrix)
    return out.reshape(B, S, D)


# speedup vs baseline: 2.3135x; 1.0213x over previous
import jax
import jax.numpy as jnp
from jax.experimental import pallas as pl
from jax.experimental.pallas import tpu as pltpu

V = 4096
D = 128
B = 8
S = 32
P = 8
CONC = 5.0
T = 32
SG = S // T
NG = B * SG
NB = 4


def _kernel(idx_ref, seg_ref, pmds_ref, pmdv_ref,
            embed_ref, kb_ref, aff_ref, lam_ref, edge_ref,
            out_ref, buf, a_scr, c_scr, g_scr, sem):
    i = pl.program_id(0)

    def issue(gi, slot):
        gb = gi // SG
        @pl.when(seg_ref[gb] > jax.lax.rem(gi, SG) * T)
        def _issue_group():
            for t in range(T):
                for p in range(P):
                    c = idx_ref[(gi * T + t) * P + p]
                    pltpu.make_async_copy(
                        edge_ref.at[pl.ds(c, 1), :],
                        buf.at[slot, p, pl.ds(t, 1), :],
                        sem.at[slot]).start()

    @pl.when(i == 0)
    def _prologue():
        for k in range(NB - 1):
            issue(k, k)
        cn_rows = []
        for bb in range(B):
            segb = seg_ref[bb]

            def body(j, acc):
                j2 = 2 * j
                c0 = idx_ref[bb * S * P + j2]
                c1 = idx_ref[bb * S * P + j2 + 1]
                w0 = pmds_ref[bb * S * P + j2]
                w1 = pmds_ref[bb * S * P + j2 + 1]
                return (acc + w0 * embed_ref[pl.ds(c0, 1), :]
                        + w1 * embed_ref[pl.ds(c1, 1), :])

            ctx = jax.lax.fori_loop(0, segb * P // 2, body,
                                    jnp.zeros((1, D), jnp.float32))
            ctx = ctx / segb.astype(jnp.float32)
            nrm = jnp.sqrt(jnp.sum(ctx * ctx))
            cn_rows.append(ctx / jnp.maximum(nrm, 1e-8))
        cnm = jnp.concatenate(cn_rows, axis=0)
        kb = kb_ref[:, :]
        dots = jax.lax.dot_general(
            cnm, kb, (((1,), (1,)), ((), ())),
            preferred_element_type=jnp.float32)
        kn2 = jax.lax.dot_general(
            jnp.ones((1, D), jnp.float32), kb * kb,
            (((1,), (1,)), ((), ())),
            preferred_element_type=jnp.float32)
        lamv = lam_ref[:, :]
        scl = (CONC * lamv) / jnp.maximum(jnp.sqrt(kn2), 1e-8)
        a_scr[:, :] = jnp.abs(dots) * scl
        c_scr[:, :] = CONC * (1.0 - lamv) * aff_ref[:, :]

    gi2 = i + NB - 1

    @pl.when(gi2 < NG)
    def _issue_ahead():
        issue(gi2, jax.lax.rem(gi2, NB))

    slot = jax.lax.rem(i, NB)
    b = i // SG
    segb = seg_ref[b]

    @pl.when(segb > jax.lax.rem(i, SG) * T)
    def _compute():
        for t in range(T):
            for p in range(P):
                pltpu.make_async_copy(
                    edge_ref.at[pl.ds(0, 1), :],
                    buf.at[slot, p, pl.ds(t, 1), :],
                    sem.at[slot]).wait()

        av = a_scr[pl.ds(b, 1), :]
        cv = c_scr[:, :]
        pmd = pmdv_ref[0]
        acc = jnp.zeros((T, V), jnp.float32)
        for p in range(P):
            rows = buf[slot, p]
            e = jnp.exp(rows * av + jnp.where(rows > 0, cv, 0.0))
            ssum = jnp.sum(e, axis=1, keepdims=True)
            colp = pmd[:, p:p + 1]
            acc = acc + e * (colp / ssum)
        g_scr[pl.ds(i * T, T), :] = acc

    @pl.when(segb <= jax.lax.rem(i, SG) * T)
    def _dead_group():
        g_scr[pl.ds(i * T, T), :] = jnp.zeros((T, V), jnp.float32)

    @pl.when(i == NG - 1)
    def _epilogue():
        out_ref[:, :] = jnp.dot(g_scr[:, :], kb_ref[:, :],
                                preferred_element_type=jnp.float32)


@jax.jit
def kernel(concepts, concepts_length, seg_len, embed, embed_kb,
           edge_matrix, affectiveness, lam):
    clen2 = concepts_length.astype(jnp.int32)
    seg = seg_len.astype(jnp.int32)

    valid = jnp.logical_and(
        jnp.arange(P)[None, None, :] < clen2[:, :, None],
        jnp.arange(S)[None, :, None] < seg[:, None, None])
    pmask = valid.astype(jnp.float32)
    pmd = (pmask / jnp.maximum(clen2, 1)[:, :, None]).reshape(-1)

    idx = concepts.astype(jnp.int32).reshape(-1)
    pmdv = pmd.reshape(NG, T, P)

    aff2 = affectiveness.reshape(1, V)
    lam2 = lam.reshape(1, V)

    full = lambda i, *_: (0, 0)

    out = pl.pallas_call(
        _kernel,
        grid_spec=pltpu.PrefetchScalarGridSpec(
            num_scalar_prefetch=3,
            grid=(NG,),
            in_specs=[
                pl.BlockSpec((1, T, P), lambda i, *_: (i, 0, 0)),
                pl.BlockSpec((V, D), full),
                pl.BlockSpec((V, D), full),
                pl.BlockSpec((1, V), full),
                pl.BlockSpec((1, V), full),
                pl.BlockSpec(memory_space=pl.ANY),
            ],
            out_specs=pl.BlockSpec((B * S, D), lambda i, *_: (0, 0)),
            scratch_shapes=[
                pltpu.VMEM((NB, P, T, V), jnp.float32),
                pltpu.VMEM((B, V), jnp.float32),
                pltpu.VMEM((1, V), jnp.float32),
                pltpu.VMEM((B * S, V), jnp.float32),
                pltpu.SemaphoreType.DMA((NB,)),
            ],
        ),
        out_shape=jax.ShapeDtypeStruct((B * S, D), jnp.float32),
    )(idx, seg, pmd, pmdv, embed, embed_kb, aff2, lam2, edge_matrix)
    return out.reshape(B, S, D)


# speedup vs baseline: 2.3143x; 1.0004x over previous
import jax
import jax.numpy as jnp
from jax.experimental import pallas as pl
from jax.experimental.pallas import tpu as pltpu

V = 4096
D = 128
B = 8
S = 32
P = 8
CONC = 5.0
T = 32
SG = S // T
NG = B * SG
NB = 6


def _kernel(idx_ref, seg_ref, pmds_ref, pmdv_ref,
            embed_ref, kb_ref, aff_ref, lam_ref, edge_ref,
            out_ref, buf, a_scr, c_scr, g_scr, sem):
    i = pl.program_id(0)

    def issue(gi, slot):
        gb = gi // SG
        @pl.when(seg_ref[gb] > jax.lax.rem(gi, SG) * T)
        def _issue_group():
            for t in range(T):
                for p in range(P):
                    c = idx_ref[(gi * T + t) * P + p]
                    pltpu.make_async_copy(
                        edge_ref.at[pl.ds(c, 1), :],
                        buf.at[slot, p, pl.ds(t, 1), :],
                        sem.at[slot]).start()

    @pl.when(i == 0)
    def _prologue():
        for k in range(NB - 1):
            issue(k, k)
        cn_rows = []
        for bb in range(B):
            segb = seg_ref[bb]

            def body(j, acc):
                j2 = 2 * j
                c0 = idx_ref[bb * S * P + j2]
                c1 = idx_ref[bb * S * P + j2 + 1]
                w0 = pmds_ref[bb * S * P + j2]
                w1 = pmds_ref[bb * S * P + j2 + 1]
                return (acc + w0 * embed_ref[pl.ds(c0, 1), :]
                        + w1 * embed_ref[pl.ds(c1, 1), :])

            ctx = jax.lax.fori_loop(0, segb * P // 2, body,
                                    jnp.zeros((1, D), jnp.float32))
            ctx = ctx / segb.astype(jnp.float32)
            nrm = jnp.sqrt(jnp.sum(ctx * ctx))
            cn_rows.append(ctx / jnp.maximum(nrm, 1e-8))
        cnm = jnp.concatenate(cn_rows, axis=0)
        kb = kb_ref[:, :]
        dots = jax.lax.dot_general(
            cnm, kb, (((1,), (1,)), ((), ())),
            preferred_element_type=jnp.float32)
        kn2 = jax.lax.dot_general(
            jnp.ones((1, D), jnp.float32), kb * kb,
            (((1,), (1,)), ((), ())),
            preferred_element_type=jnp.float32)
        lamv = lam_ref[:, :]
        scl = (CONC * lamv) / jnp.maximum(jnp.sqrt(kn2), 1e-8)
        a_scr[:, :] = jnp.abs(dots) * scl
        c_scr[:, :] = CONC * (1.0 - lamv) * aff_ref[:, :]

    gi2 = i + NB - 1

    @pl.when(gi2 < NG)
    def _issue_ahead():
        issue(gi2, jax.lax.rem(gi2, NB))

    slot = jax.lax.rem(i, NB)
    b = i // SG
    segb = seg_ref[b]

    @pl.when(segb > jax.lax.rem(i, SG) * T)
    def _compute():
        for t in range(T):
            for p in range(P):
                pltpu.make_async_copy(
                    edge_ref.at[pl.ds(0, 1), :],
                    buf.at[slot, p, pl.ds(t, 1), :],
                    sem.at[slot]).wait()

        av = a_scr[pl.ds(b, 1), :]
        cv = c_scr[:, :]
        pmd = pmdv_ref[0]
        acc = jnp.zeros((T, V), jnp.float32)
        for p in range(P):
            rows = buf[slot, p]
            e = jnp.exp(rows * av + jnp.where(rows > 0, cv, 0.0))
            ssum = jnp.sum(e, axis=1, keepdims=True)
            colp = pmd[:, p:p + 1]
            acc = acc + e * (colp / ssum)
        g_scr[pl.ds(i * T, T), :] = acc

    @pl.when(segb <= jax.lax.rem(i, SG) * T)
    def _dead_group():
        g_scr[pl.ds(i * T, T), :] = jnp.zeros((T, V), jnp.float32)

    @pl.when(i == NG - 1)
    def _epilogue():
        out_ref[:, :] = jnp.dot(g_scr[:, :], kb_ref[:, :],
                                preferred_element_type=jnp.float32)


@jax.jit
def kernel(concepts, concepts_length, seg_len, embed, embed_kb,
           edge_matrix, affectiveness, lam):
    clen2 = concepts_length.astype(jnp.int32)
    seg = seg_len.astype(jnp.int32)

    valid = jnp.logical_and(
        jnp.arange(P)[None, None, :] < clen2[:, :, None],
        jnp.arange(S)[None, :, None] < seg[:, None, None])
    pmask = valid.astype(jnp.float32)
    pmd = (pmask / jnp.maximum(clen2, 1)[:, :, None]).reshape(-1)

    idx = concepts.astype(jnp.int32).reshape(-1)
    pmdv = pmd.reshape(NG, T, P)

    aff2 = affectiveness.reshape(1, V)
    lam2 = lam.reshape(1, V)

    full = lambda i, *_: (0, 0)

    out = pl.pallas_call(
        _kernel,
        grid_spec=pltpu.PrefetchScalarGridSpec(
            num_scalar_prefetch=3,
            grid=(NG,),
            in_specs=[
                pl.BlockSpec((1, T, P), lambda i, *_: (i, 0, 0)),
                pl.BlockSpec((V, D), full),
                pl.BlockSpec((V, D), full),
                pl.BlockSpec((1, V), full),
                pl.BlockSpec((1, V), full),
                pl.BlockSpec(memory_space=pl.ANY),
            ],
            out_specs=pl.BlockSpec((B * S, D), lambda i, *_: (0, 0)),
            scratch_shapes=[
                pltpu.VMEM((NB, P, T, V), jnp.float32),
                pltpu.VMEM((B, V), jnp.float32),
                pltpu.VMEM((1, V), jnp.float32),
                pltpu.VMEM((B * S, V), jnp.float32),
                pltpu.SemaphoreType.DMA((NB,)),
            ],
        ),
        out_shape=jax.ShapeDtypeStruct((B * S, D), jnp.float32),
    )(idx, seg, pmd, pmdv, embed, embed_kb, aff2, lam2, edge_matrix)
    return out.reshape(B, S, D)
